# baseline (speedup 1.0000x reference)
import functools

import jax
import jax.numpy as jnp
import numpy as np
from jax import lax
from jax.experimental import pallas as pl
from jax.experimental.pallas import tpu as pltpu
from jax.experimental.pallas import tpu_sc as plsc

N = 10000
E = 320000
IN = 128
HID = 128
ED = 16
G = 64

NC = 2
NS = 16
NW = NC * NS
CHUNK = 80
CPT = E // (NW * CHUNK)
N_STRIPE = 640
N_PAD = NS * N_STRIPE

PACK = 10
EP = E // PACK
BR = 800


def _ea_body(a_ref, w_ref, b_ref, o_ref):
    o_ref[...] = (
        jnp.dot(a_ref[...], w_ref[...], preferred_element_type=jnp.float32)
        + b_ref[...]
    )


def _ea_call(ap, w_big, b_big):
    return pl.pallas_call(
        _ea_body,
        grid=(EP // BR,),
        in_specs=[
            pl.BlockSpec((BR, PACK * ED), lambda e: (e, 0)),
            pl.BlockSpec((PACK * ED, PACK * HID), lambda e: (0, 0)),
            pl.BlockSpec((1, PACK * HID), lambda e: (0, 0)),
        ],
        out_specs=pl.BlockSpec((BR, PACK * HID), lambda e: (e, 0)),
        out_shape=jax.ShapeDtypeStruct((EP, PACK * HID), jnp.float32),
    )(ap, w_big, b_big)


def _sc_body(h_hbm, ea_hbm, idx_hbm, out_hbm,
             idx_v, hb0, eb0, hb1, eb1, agg,
             si0, si1, sg0, sg1, se0, se1):
    c = lax.axis_index("c")
    s = lax.axis_index("s")
    wid = c * NS + s
    hb = (hb0, hb1)
    eb = (eb0, eb1)
    sem_i = (si0, si1)
    sem_g = (sg0, sg1)
    sem_e = (se0, se1)

    def idx_cp(ci, k):
        return pltpu.make_async_copy(idx_hbm.at[wid, ci], idx_v.at[k],
                                     sem_i[k])

    def data_cp(ci, k):
        gcp = pltpu.make_async_copy(h_hbm.at[idx_v.at[k, 0]], hb[k],
                                    sem_g[k])
        baser = (wid * CPT + ci) * (CHUNK // PACK)
        ecp = pltpu.make_async_copy(
            ea_hbm.at[pl.ds(baser, CHUNK // PACK)], eb[k], sem_e[k])
        return gcp, ecp

    def compute_scatter(b):
        @pl.loop(0, CHUNK // PACK, unroll=4)
        def _ro(ro):
            for ri in range(PACK):
                r = ro * PACK + ri
                for kk in range(HID // 16):
                    sl = pl.ds(kk * 16, 16)
                    esl = pl.ds(ri * HID + kk * 16, 16)
                    hb[b][r, sl] = jnp.maximum(
                        hb[b][r, sl] + eb[b][ro, esl], 0.0)

        pltpu.sync_copy(hb[b], agg.at[idx_v.at[b, 1]], add=True)

    @pl.loop(0, CHUNK)
    def _zrow(r):
        for k in range(HID // 16):
            hb0[r, pl.ds(k * 16, 16)] = jnp.zeros((16,), jnp.float32)

    @pl.loop(0, N_STRIPE // CHUNK)
    def _zcp(j):
        pltpu.sync_copy(hb0, agg.at[pl.ds(s * N_STRIPE + j * CHUNK, CHUNK)])

    plsc.subcore_barrier()

    idx_cp(0, 0).start()
    idx_cp(1, 1).start()
    idx_cp(0, 0).wait()
    g0, e0 = data_cp(0, 0)
    g0.start()
    e0.start()

    @pl.loop(0, CPT - 1, step=2)
    def _edge(i0):
        for u in range(2):
            i = i0 + u
            b = u
            ob = 1 - u

            idx_cp(i + 1, ob).wait()
            gn, en = data_cp(i + 1, ob)
            gn.start()
            en.start()

            gc, ec = data_cp(i, b)
            gc.wait()
            ec.wait()

            compute_scatter(b)

            @pl.when(i + 2 < CPT)
            def _pref_idx():
                idx_cp(i + 2, b).start()

    gl, el = data_cp(CPT - 1, 0)
    gl.wait()
    el.wait()
    compute_scatter(0)

    plsc.subcore_barrier()

    @pl.loop(0, N_STRIPE // CHUNK)
    def _drain(j):
        row0 = s * N_STRIPE + j * CHUNK
        pltpu.sync_copy(agg.at[pl.ds(row0, CHUNK)],
                        out_hbm.at[c, pl.ds(row0, CHUNK)])


def _sc_call(h, ea_l, idx_p):
    mesh = plsc.VectorSubcoreMesh(core_axis_name="c", subcore_axis_name="s")
    kfn = pl.kernel(
        _sc_body,
        out_type=jax.ShapeDtypeStruct((NC, N_PAD, HID), jnp.float32),
        mesh=mesh,
        scratch_types=[
            pltpu.VMEM((2, 2, CHUNK), jnp.int32),
            pltpu.VMEM((CHUNK, HID), jnp.float32),
            pltpu.VMEM((CHUNK // PACK, PACK * HID), jnp.float32),
            pltpu.VMEM((CHUNK, HID), jnp.float32),
            pltpu.VMEM((CHUNK // PACK, PACK * HID), jnp.float32),
            pltpu.VMEM_SHARED((N_PAD, HID), jnp.float32),
            pltpu.SemaphoreType.DMA,
            pltpu.SemaphoreType.DMA,
            pltpu.SemaphoreType.DMA,
            pltpu.SemaphoreType.DMA,
            pltpu.SemaphoreType.DMA,
            pltpu.SemaphoreType.DMA,
        ],
    )
    return kfn(h, ea_l, idx_p)


def _dense_body(h_ref, a_ref, w_ref, b_ref, o_ref):
    t = h_ref[...] + a_ref[0, :N, :] + a_ref[1, :N, :]
    o_ref[...] = jnp.maximum(
        jnp.dot(t, w_ref[...], preferred_element_type=jnp.float32)
        + b_ref[...],
        0.0,
    )


def _dense_call(h, agg, w, b):
    return pl.pallas_call(
        _dense_body,
        out_shape=jax.ShapeDtypeStruct((N, HID), jnp.float32),
    )(h, agg, w, b)


def _pool_body(h_ref, batch_ref, w_ref, b_ref, o_ref):
    gid = lax.broadcasted_iota(jnp.int32, (G, 1), 0)
    pt = (batch_ref[...] == gid).astype(jnp.float32)
    sums = jnp.dot(pt, h_ref[...], preferred_element_type=jnp.float32)
    counts = jnp.sum(pt, axis=1, keepdims=True)
    pooled = sums / jnp.maximum(counts, 1.0)
    o_ref[...] = (
        jnp.dot(pooled, w_ref[...], preferred_element_type=jnp.float32)
        + b_ref[...]
    )


def _pool_call(h, batch2d, w, b):
    return pl.pallas_call(
        _pool_body,
        out_shape=jax.ShapeDtypeStruct((G, HID), jnp.float32),
    )(h, batch2d, w, b)


def kernel(x, edge_index, edge_attr, batch,
           We1, be1, W1, b1,
           We2, be2, W2, b2,
           We3, be3, W3, b3,
           Wlin, blin):
    src = edge_index[0].astype(jnp.int32)
    dst = edge_index[1].astype(jnp.int32)
    idx_p = jnp.stack([src.reshape(NW, CPT, CHUNK),
                       dst.reshape(NW, CPT, CHUNK)], axis=2)

    eyep = jnp.eye(PACK, dtype=jnp.float32)
    w_cat = jnp.stack([We1, We2, We3])
    w_big = jnp.einsum("ij,lkc->likjc", eyep, w_cat) \
        .reshape(3, PACK * ED, PACK * HID)
    b_big = jnp.concatenate(
        [jnp.tile(be1, PACK), jnp.tile(be2, PACK), jnp.tile(be3, PACK)]
    ).reshape(3, 1, PACK * HID)
    ap = edge_attr.reshape(EP, PACK * ED)

    ea = [_ea_call(ap, w_big[layer], b_big[layer])
          for layer in range(3)]

    h = x
    for layer, (w, b) in enumerate(((W1, b1), (W2, b2), (W3, b3))):
        agg = _sc_call(h, ea[layer], idx_p)
        h = _dense_call(h, agg, w, b.reshape(1, HID))

    return _pool_call(h, batch.astype(jnp.int32).reshape(1, N),
                      Wlin, blin.reshape(1, HID))


# speedup vs baseline: 1.3408x; 1.3408x over previous
import functools

import jax
import jax.numpy as jnp
import numpy as np
from jax import lax
from jax.experimental import pallas as pl
from jax.experimental.pallas import tpu as pltpu
from jax.experimental.pallas import tpu_sc as plsc

N = 10000
E = 320000
IN = 128
HID = 128
ED = 16
G = 64

NC = 2
NS = 16
NW = NC * NS
CHUNK = 80
CPT = E // (NW * CHUNK)
N_STRIPE = 640
N_PAD = NS * N_STRIPE

PACK = 10
EP = E // PACK
BR = 800


def _ea_body(a_ref, w_ref, b_ref, o_ref):
    o_ref[...] = (
        jnp.dot(a_ref[...], w_ref[...], preferred_element_type=jnp.float32)
        + b_ref[...]
    )


def _ea_call(ap, w_big, b_big):
    return pl.pallas_call(
        _ea_body,
        grid=(EP // BR,),
        in_specs=[
            pl.BlockSpec((BR, PACK * ED), lambda e: (e, 0)),
            pl.BlockSpec((PACK * ED, PACK * HID), lambda e: (0, 0)),
            pl.BlockSpec((1, PACK * HID), lambda e: (0, 0)),
        ],
        out_specs=pl.BlockSpec((BR, PACK * HID), lambda e: (e, 0)),
        out_shape=jax.ShapeDtypeStruct((EP, PACK * HID), jnp.float32),
    )(ap, w_big, b_big)


def _sc_body(h_hbm, ea_hbm, idx_hbm, out_hbm,
             idx_v, hb0, eb0, hb1, eb1, agg,
             si0, si1, sg0, sg1, se0, se1):
    c = lax.axis_index("c")
    s = lax.axis_index("s")
    wid = c * NS + s
    hb = (hb0, hb1)
    eb = (eb0, eb1)
    sem_i = (si0, si1)
    sem_g = (sg0, sg1)
    sem_e = (se0, se1)

    def idx_cp(ci, k):
        return pltpu.make_async_copy(idx_hbm.at[wid, ci], idx_v.at[k],
                                     sem_i[k])

    def data_cp(ci, k):
        gcp = pltpu.make_async_copy(h_hbm.at[idx_v.at[k, 0]], hb[k],
                                    sem_g[k])
        baser = (wid * CPT + ci) * (CHUNK // PACK)
        ecp = pltpu.make_async_copy(
            ea_hbm.at[pl.ds(baser, CHUNK // PACK)], eb[k], sem_e[k])
        return gcp, ecp

    def compute_scatter(b):
        @pl.loop(0, CHUNK // PACK, unroll=2)
        def _ro(ro):
            for ri in range(PACK):
                r = ro * PACK + ri
                for kk in range(HID // 16):
                    sl = pl.ds(kk * 16, 16)
                    esl = pl.ds(ri * HID + kk * 16, 16)
                    hb[b][r, sl] = jnp.maximum(
                        hb[b][r, sl] + eb[b][ro, esl], 0.0)

        pltpu.sync_copy(hb[b], agg.at[idx_v.at[b, 1]], add=True)

    @pl.loop(0, CHUNK)
    def _zrow(r):
        for k in range(HID // 16):
            hb0[r, pl.ds(k * 16, 16)] = jnp.zeros((16,), jnp.float32)

    @pl.loop(0, N_STRIPE // CHUNK)
    def _zcp(j):
        pltpu.sync_copy(hb0, agg.at[pl.ds(s * N_STRIPE + j * CHUNK, CHUNK)])

    plsc.subcore_barrier()

    idx_cp(0, 0).start()
    idx_cp(1, 1).start()
    idx_cp(0, 0).wait()
    g0, e0 = data_cp(0, 0)
    g0.start()
    e0.start()

    @pl.loop(0, CPT - 1, step=2)
    def _edge(i0):
        for u in range(2):
            i = i0 + u
            b = u
            ob = 1 - u

            idx_cp(i + 1, ob).wait()
            gn, en = data_cp(i + 1, ob)
            gn.start()
            en.start()

            gc, ec = data_cp(i, b)
            gc.wait()
            ec.wait()

            compute_scatter(b)

            @pl.when(i + 2 < CPT)
            def _pref_idx():
                idx_cp(i + 2, b).start()

    gl, el = data_cp(CPT - 1, 0)
    gl.wait()
    el.wait()
    compute_scatter(0)

    plsc.subcore_barrier()

    @pl.loop(0, N_STRIPE // CHUNK)
    def _drain(j):
        row0 = s * N_STRIPE + j * CHUNK
        pltpu.sync_copy(agg.at[pl.ds(row0, CHUNK)],
                        out_hbm.at[c, pl.ds(row0, CHUNK)])


def _sc_call(h, ea_l, idx_p):
    mesh = plsc.VectorSubcoreMesh(core_axis_name="c", subcore_axis_name="s")
    kfn = pl.kernel(
        _sc_body,
        out_type=jax.ShapeDtypeStruct((NC, N_PAD, HID), jnp.float32),
        mesh=mesh,
        scratch_types=[
            pltpu.VMEM((2, 2, CHUNK), jnp.int32),
            pltpu.VMEM((CHUNK, HID), jnp.float32),
            pltpu.VMEM((CHUNK // PACK, PACK * HID), jnp.float32),
            pltpu.VMEM((CHUNK, HID), jnp.float32),
            pltpu.VMEM((CHUNK // PACK, PACK * HID), jnp.float32),
            pltpu.VMEM_SHARED((N_PAD, HID), jnp.float32),
            pltpu.SemaphoreType.DMA,
            pltpu.SemaphoreType.DMA,
            pltpu.SemaphoreType.DMA,
            pltpu.SemaphoreType.DMA,
            pltpu.SemaphoreType.DMA,
            pltpu.SemaphoreType.DMA,
        ],
    )
    return kfn(h, ea_l, idx_p)


def _dense_body(h_ref, a_ref, w_ref, b_ref, o_ref):
    t = h_ref[...] + a_ref[0, :N, :] + a_ref[1, :N, :]
    o_ref[...] = jnp.maximum(
        jnp.dot(t, w_ref[...], preferred_element_type=jnp.float32)
        + b_ref[...],
        0.0,
    )


def _dense_call(h, agg, w, b):
    return pl.pallas_call(
        _dense_body,
        out_shape=jax.ShapeDtypeStruct((N, HID), jnp.float32),
    )(h, agg, w, b)


def _pool_body(h_ref, batch_ref, w_ref, b_ref, o_ref):
    gid = lax.broadcasted_iota(jnp.int32, (G, 1), 0)
    pt = (batch_ref[...] == gid).astype(jnp.float32)
    sums = jnp.dot(pt, h_ref[...], preferred_element_type=jnp.float32)
    counts = jnp.sum(pt, axis=1, keepdims=True)
    pooled = sums / jnp.maximum(counts, 1.0)
    o_ref[...] = (
        jnp.dot(pooled, w_ref[...], preferred_element_type=jnp.float32)
        + b_ref[...]
    )


def _pool_call(h, batch2d, w, b):
    return pl.pallas_call(
        _pool_body,
        out_shape=jax.ShapeDtypeStruct((G, HID), jnp.float32),
    )(h, batch2d, w, b)


def kernel(x, edge_index, edge_attr, batch,
           We1, be1, W1, b1,
           We2, be2, W2, b2,
           We3, be3, W3, b3,
           Wlin, blin):
    src = edge_index[0].astype(jnp.int32)
    dst = edge_index[1].astype(jnp.int32)
    idx_p = jnp.stack([src.reshape(NW, CPT, CHUNK),
                       dst.reshape(NW, CPT, CHUNK)], axis=2)

    eyep = jnp.eye(PACK, dtype=jnp.float32)
    w_cat = jnp.stack([We1, We2, We3])
    w_big = jnp.einsum("ij,lkc->likjc", eyep, w_cat) \
        .reshape(3, PACK * ED, PACK * HID)
    b_big = jnp.concatenate(
        [jnp.tile(be1, PACK), jnp.tile(be2, PACK), jnp.tile(be3, PACK)]
    ).reshape(3, 1, PACK * HID)
    ap = edge_attr.reshape(EP, PACK * ED)

    ea = [_ea_call(ap, w_big[layer], b_big[layer])
          for layer in range(3)]

    h = x
    for layer, (w, b) in enumerate(((W1, b1), (W2, b2), (W3, b3))):
        agg = _sc_call(h, ea[layer], idx_p)
        h = _dense_call(h, agg, w, b.reshape(1, HID))

    return _pool_call(h, batch.astype(jnp.int32).reshape(1, N),
                      Wlin, blin.reshape(1, HID))
